# baseline (speedup 1.0000x reference)
import functools

import jax
import jax.numpy as jnp
from jax import lax
from jax.experimental import pallas as pl
from jax.experimental.pallas import tpu as pltpu
from jax.experimental.pallas import tpu_sc as plsc

N = 10000
D = 128
H = 128
E = 320000
NL = 40
MH = 256

NC = 2
NS = 16
NW = NC * NS
B = 128
NBW = 80
CH = 16
NCH = NBW // CH
EP = NW * NBW * B
NA = 10112
RPT = NA // NS
ZCH = (128, 128, 128, 128, 120)
IDP = 1024

_mesh = plsc.VectorSubcoreMesh(
    core_axis_name="c", subcore_axis_name="s", num_cores=NC, num_subcores=NS
)


def _fill_f32(buf, rows, cols, val):
    vec = jnp.full((16,), val, jnp.float32)

    def body(i, carry):
        for k in range(cols // 16):
            buf[i, pl.ds(k * 16, 16)] = vec
        return carry

    lax.fori_loop(0, rows, body, 0)


@functools.partial(
    pl.kernel,
    out_type=jax.ShapeDtypeStruct((2 * NA, H), jnp.float32),
    mesh=_mesh,
    scratch_types=[
        pltpu.VMEM((CH, B), jnp.int32),
        pltpu.VMEM((CH, B), jnp.int32),
        pltpu.VMEM((B, H), jnp.float32),
        pltpu.VMEM((B, H), jnp.float32),
        pltpu.VMEM_SHARED((NA, H), jnp.float32),
        pltpu.SemaphoreType.DMA,
        pltpu.SemaphoreType.DMA,
    ],
)
def _agg_pass(h_hbm, srcp_hbm, dstp_hbm, out_hbm, sidx_v, didx_v, rows_a, rows_b,
              agg_sh, gsem, ssem):
    c = lax.axis_index("c")
    s = lax.axis_index("s")
    wid = c * NS + s

    _fill_f32(rows_a, B, H, 0.0)
    off = 0
    for sz in ZCH:
        pltpu.sync_copy(rows_a.at[pl.ds(0, sz)], agg_sh.at[pl.ds(s * RPT + off, sz)])
        off += sz
    plsc.subcore_barrier()

    def gather(j, buf):
        pltpu.async_copy(h_hbm.at[sidx_v.at[j]], buf, gsem)

    def gather_wait(buf):
        pltpu.make_async_copy(h_hbm.at[pl.ds(0, B)], buf, gsem).wait()

    def scatter(j, buf):
        pltpu.async_copy(buf, agg_sh.at[didx_v.at[j]], ssem, add=True)

    def scatter_wait(buf):
        pltpu.make_async_copy(buf, agg_sh.at[pl.ds(0, B)], ssem).wait()

    bufs = (rows_a, rows_b)

    def chunk_body(ch, carry):
        base = wid * NBW + ch * CH
        pltpu.sync_copy(srcp_hbm.at[pl.ds(base, CH)], sidx_v)
        pltpu.sync_copy(dstp_hbm.at[pl.ds(base, CH)], didx_v)
        gather(0, bufs[0])
        for jj in range(1, CH):
            cur, prv = bufs[jj % 2], bufs[(jj - 1) % 2]
            if jj >= 2:
                scatter_wait(cur)
            gather(jj, cur)
            gather_wait(prv)
            scatter(jj - 1, prv)
        last = bufs[(CH - 1) % 2]
        scatter_wait(bufs[(CH - 2) % 2])
        gather_wait(last)
        scatter(CH - 1, last)
        scatter_wait(last)
        return carry

    lax.fori_loop(0, NCH, chunk_body, 0)
    plsc.subcore_barrier()

    pltpu.sync_copy(
        agg_sh.at[pl.ds(s * RPT, RPT)], out_hbm.at[pl.ds(c * NA + s * RPT, RPT)]
    )


@functools.partial(
    pl.kernel,
    out_type=jax.ShapeDtypeStruct((2 * NA, H), jnp.float32),
    mesh=_mesh,
    scratch_types=[
        pltpu.VMEM((CH, B), jnp.int32),
        pltpu.VMEM((B, H), jnp.float32),
        pltpu.VMEM_SHARED((NA, H), jnp.float32),
        pltpu.SemaphoreType.DMA,
    ],
)
def _deg_pass(dstp_hbm, out_hbm, didx_v, rows_v, deg_sh, ssem):
    c = lax.axis_index("c")
    s = lax.axis_index("s")
    wid = c * NS + s

    _fill_f32(rows_v, B, H, 0.0)
    off = 0
    for sz in ZCH:
        pltpu.sync_copy(rows_v.at[pl.ds(0, sz)], deg_sh.at[pl.ds(s * RPT + off, sz)])
        off += sz
    _fill_f32(rows_v, B, H, 1.0)
    plsc.subcore_barrier()

    def chunk_body(ch, carry):
        pltpu.sync_copy(dstp_hbm.at[pl.ds(wid * NBW + ch * CH, CH)], didx_v)
        for jj in range(CH):
            pltpu.async_copy(rows_v, deg_sh.at[didx_v.at[jj]], ssem, add=True)
        for jj in range(CH):
            pltpu.make_async_copy(rows_v, deg_sh.at[pl.ds(0, B)], ssem).wait()
        return carry

    lax.fori_loop(0, NCH, chunk_body, 0)
    plsc.subcore_barrier()
    pltpu.sync_copy(
        deg_sh.at[pl.ds(s * RPT, RPT)], out_hbm.at[pl.ds(c * NA + s * RPT, RPT)]
    )


BN = 1000


def _compress_body(dw_ref, ids_ref, invd_ref, msk_ref):
    i = pl.program_id(0)
    d = jnp.maximum(dw_ref[0, :, 0:1] + dw_ref[1, :, 0:1], 1.0)
    invd_ref[...] = 1.0 / d
    rowid = jax.lax.broadcasted_iota(jnp.int32, (BN, IDP), 0) + i * BN
    hit = rowid == ids_ref[...]
    msk_ref[...] = jnp.any(hit, axis=1, keepdims=True).astype(jnp.float32)


_compress_tc = pl.pallas_call(
    _compress_body,
    grid=(N // BN,),
    in_specs=[
        pl.BlockSpec((2, BN, H), lambda i: (0, i, 0)),
        pl.BlockSpec((1, IDP), lambda i: (0, 0)),
    ],
    out_specs=[
        pl.BlockSpec((BN, 1), lambda i: (i, 0)),
        pl.BlockSpec((BN, 1), lambda i: (i, 0)),
    ],
    out_shape=[
        jax.ShapeDtypeStruct((N, 1), jnp.float32),
        jax.ShapeDtypeStruct((N, 1), jnp.float32),
    ],
)


def _sage_block(h, agg_ref, invd_ref, msk_ref, ws_ref, wn_ref, wi_ref, b_ref):
    agg = agg_ref[0] + agg_ref[1]
    neigh = agg * invd_ref[...]
    acc = (
        jnp.dot(h, ws_ref[...], preferred_element_type=jnp.float32)
        + jnp.dot(neigh, wn_ref[...], preferred_element_type=jnp.float32)
        + msk_ref[...] * jnp.dot(h, wi_ref[...], preferred_element_type=jnp.float32)
        + b_ref[...]
    )
    return jnp.maximum(acc, 0.0)


def _layer_body(h_ref, agg_ref, invd_ref, msk_ref, ws_ref, wn_ref, wi_ref, b_ref, o_ref):
    o_ref[...] = _sage_block(
        h_ref[...], agg_ref, invd_ref, msk_ref, ws_ref, wn_ref, wi_ref, b_ref
    )


def _head_body(h_ref, agg_ref, invd_ref, msk_ref, ws_ref, wn_ref, wi_ref, b_ref,
               wm1_ref, bm1_ref, wm2_ref, bm2_ref, o_ref):
    h3 = _sage_block(
        h_ref[...], agg_ref, invd_ref, msk_ref, ws_ref, wn_ref, wi_ref, b_ref
    )
    t = jnp.maximum(
        jnp.dot(h3, wm1_ref[...], preferred_element_type=jnp.float32) + bm1_ref[...],
        0.0,
    )
    o_ref[...] = jnp.dot(t, wm2_ref[...], preferred_element_type=jnp.float32) + bm2_ref[...]


_node_specs = [
    pl.BlockSpec((BN, H), lambda i: (i, 0)),
    pl.BlockSpec((2, BN, H), lambda i: (0, i, 0)),
    pl.BlockSpec((BN, 1), lambda i: (i, 0)),
    pl.BlockSpec((BN, 1), lambda i: (i, 0)),
]
_w_specs = [
    pl.BlockSpec((D, H), lambda i: (0, 0)),
    pl.BlockSpec((D, H), lambda i: (0, 0)),
    pl.BlockSpec((D, H), lambda i: (0, 0)),
    pl.BlockSpec((1, H), lambda i: (0, 0)),
]

_layer_tc = pl.pallas_call(
    _layer_body,
    grid=(N // BN,),
    in_specs=_node_specs + _w_specs,
    out_specs=pl.BlockSpec((BN, H), lambda i: (i, 0)),
    out_shape=jax.ShapeDtypeStruct((N, H), jnp.float32),
)

_head_tc = pl.pallas_call(
    _head_body,
    grid=(N // BN,),
    in_specs=_node_specs + _w_specs + [
        pl.BlockSpec((H, MH), lambda i: (0, 0)),
        pl.BlockSpec((1, MH), lambda i: (0, 0)),
        pl.BlockSpec((MH, NL), lambda i: (0, 0)),
        pl.BlockSpec((1, NL), lambda i: (0, 0)),
    ],
    out_specs=pl.BlockSpec((BN, NL), lambda i: (i, 0)),
    out_shape=jax.ShapeDtypeStruct((N, NL), jnp.float32),
)


def kernel(x, unused, Ws0, Wn0, Wi0, b0, Ws1, Wn1, Wi1, b1, Ws2, Wn2, Wi2, b2,
           Wm1, bm1, Wm2, bm2, edge_index, id_index):
    src = edge_index[0].astype(jnp.int32)
    dst = edge_index[1].astype(jnp.int32)
    srcp = jnp.concatenate([src, jnp.zeros((EP - E,), jnp.int32)]).reshape(NW * NBW, B)
    dstp = jnp.concatenate([dst, jnp.full((EP - E,), N, jnp.int32)]).reshape(NW * NBW, B)
    idp = jnp.concatenate(
        [id_index.astype(jnp.int32), jnp.full((IDP - id_index.shape[0],), N, jnp.int32)]
    )

    degw = _deg_pass(dstp).reshape(2, NA, H)[:, :N, :]
    invd, msk = _compress_tc(degw, idp.reshape(1, IDP))

    b0r = b0.reshape(1, H)
    b1r = b1.reshape(1, H)
    b2r = b2.reshape(1, H)
    bm1r = bm1.reshape(1, MH)
    bm2r = bm2.reshape(1, NL)

    h = x
    agg = _agg_pass(h, srcp, dstp).reshape(2, NA, H)[:, :N, :]
    h = _layer_tc(h, agg, invd, msk, Ws0, Wn0, Wi0, b0r)
    agg = _agg_pass(h, srcp, dstp).reshape(2, NA, H)[:, :N, :]
    h = _layer_tc(h, agg, invd, msk, Ws1, Wn1, Wi1, b1r)
    agg = _agg_pass(h, srcp, dstp).reshape(2, NA, H)[:, :N, :]
    return _head_tc(h, agg, invd, msk, Ws2, Wn2, Wi2, b2r, Wm1, bm1r, Wm2, bm2r)


# speedup vs baseline: 2.9370x; 2.9370x over previous
import functools

import jax
import jax.numpy as jnp
from jax import lax
from jax.experimental import pallas as pl
from jax.experimental.pallas import tpu as pltpu
from jax.experimental.pallas import tpu_sc as plsc

N = 10000
D = 128
H = 128
E = 320000
NL = 40
MH = 256

NC = 2
NS = 16
NW = NC * NS
B = 128
NBW = 80
CH = 16
NCH = NBW // CH
EP = NW * NBW * B
NA = 10112
RPT = NA // NS
ZCH = (128, 128, 128, 128, 120)
IDP = 1024

_mesh = plsc.VectorSubcoreMesh(
    core_axis_name="c", subcore_axis_name="s", num_cores=NC, num_subcores=NS
)


def _fill_f32(buf, rows, cols, val):
    vec = jnp.full((16,), val, jnp.float32)

    def body(i, carry):
        for k in range(cols // 16):
            buf[i, pl.ds(k * 16, 16)] = vec
        return carry

    lax.fori_loop(0, rows, body, 0)


@functools.partial(
    pl.kernel,
    out_type=jax.ShapeDtypeStruct((2 * NA, H), jnp.float32),
    mesh=_mesh,
    scratch_types=[
        pltpu.VMEM((CH, B), jnp.int32),
        pltpu.VMEM((CH, B), jnp.int32),
        pltpu.VMEM((B, H), jnp.float32),
        pltpu.VMEM((B, H), jnp.float32),
        pltpu.VMEM_SHARED((NA, H), jnp.float32),
        pltpu.SemaphoreType.DMA,
        pltpu.SemaphoreType.DMA,
    ],
)
def _agg_pass(h_hbm, srcp_hbm, dstp_hbm, out_hbm, sidx_v, didx_v, rows_a, rows_b,
              agg_sh, gsem, ssem):
    c = lax.axis_index("c")
    s = lax.axis_index("s")
    wid = c * NS + s

    _fill_f32(rows_a, B, H, 0.0)
    off = 0
    for sz in ZCH:
        pltpu.sync_copy(rows_a.at[pl.ds(0, sz)], agg_sh.at[pl.ds(s * RPT + off, sz)])
        off += sz
    plsc.subcore_barrier()

    def gather(j, buf):
        pltpu.async_copy(h_hbm.at[sidx_v.at[j]], buf, gsem)

    def gather_wait(buf):
        pltpu.make_async_copy(h_hbm.at[pl.ds(0, B)], buf, gsem).wait()

    def scatter(j, buf):
        pltpu.async_copy(buf, agg_sh.at[didx_v.at[j]], ssem, add=True)

    def scatter_wait(buf):
        pltpu.make_async_copy(buf, agg_sh.at[pl.ds(0, B)], ssem).wait()

    bufs = (rows_a, rows_b)

    def chunk_body(ch, carry):
        base = wid * NBW + ch * CH
        pltpu.sync_copy(srcp_hbm.at[pl.ds(base, CH)], sidx_v)
        pltpu.sync_copy(dstp_hbm.at[pl.ds(base, CH)], didx_v)
        gather(0, bufs[0])
        for jj in range(1, CH):
            cur, prv = bufs[jj % 2], bufs[(jj - 1) % 2]
            if jj >= 2:
                scatter_wait(cur)
            gather(jj, cur)
            gather_wait(prv)
            scatter(jj - 1, prv)
        last = bufs[(CH - 1) % 2]
        scatter_wait(bufs[(CH - 2) % 2])
        gather_wait(last)
        scatter(CH - 1, last)
        scatter_wait(last)
        return carry

    lax.fori_loop(0, NCH, chunk_body, 0)
    plsc.subcore_barrier()

    pltpu.sync_copy(
        agg_sh.at[pl.ds(s * RPT, RPT)], out_hbm.at[pl.ds(c * NA + s * RPT, RPT)]
    )


@functools.partial(
    pl.kernel,
    out_type=jax.ShapeDtypeStruct((2 * NA, H), jnp.float32),
    mesh=_mesh,
    scratch_types=[
        pltpu.VMEM((CH, B), jnp.int32),
        pltpu.VMEM((B, H), jnp.float32),
        pltpu.VMEM_SHARED((NA, H), jnp.float32),
        pltpu.SemaphoreType.DMA,
    ],
)
def _deg_pass(dstp_hbm, out_hbm, didx_v, rows_v, deg_sh, ssem):
    c = lax.axis_index("c")
    s = lax.axis_index("s")
    wid = c * NS + s

    _fill_f32(rows_v, B, H, 0.0)
    off = 0
    for sz in ZCH:
        pltpu.sync_copy(rows_v.at[pl.ds(0, sz)], deg_sh.at[pl.ds(s * RPT + off, sz)])
        off += sz
    _fill_f32(rows_v, B, H, 1.0)
    plsc.subcore_barrier()

    def chunk_body(ch, carry):
        pltpu.sync_copy(dstp_hbm.at[pl.ds(wid * NBW + ch * CH, CH)], didx_v)
        for jj in range(CH):
            pltpu.async_copy(rows_v, deg_sh.at[didx_v.at[jj]], ssem, add=True)
        for jj in range(CH):
            pltpu.make_async_copy(rows_v, deg_sh.at[pl.ds(0, B)], ssem).wait()
        return carry

    lax.fori_loop(0, NCH, chunk_body, 0)
    plsc.subcore_barrier()
    pltpu.sync_copy(
        deg_sh.at[pl.ds(s * RPT, RPT)], out_hbm.at[pl.ds(c * NA + s * RPT, RPT)]
    )


BN = 1000


def _compress_body(dw_ref, ids_ref, invd_ref, msk_ref):
    i = pl.program_id(0)
    d = jnp.maximum(dw_ref[0, :, 0:1] + dw_ref[1, :, 0:1], 1.0)
    invd_ref[...] = 1.0 / d
    rowid = jax.lax.broadcasted_iota(jnp.int32, (BN, IDP), 0) + i * BN
    hit = rowid == ids_ref[...]
    msk_ref[...] = jnp.any(hit, axis=1, keepdims=True).astype(jnp.float32)


_compress_tc = pl.pallas_call(
    _compress_body,
    grid=(N // BN,),
    in_specs=[
        pl.BlockSpec((2, BN, H), lambda i: (0, i, 0)),
        pl.BlockSpec((1, IDP), lambda i: (0, 0)),
    ],
    out_specs=[
        pl.BlockSpec((BN, 1), lambda i: (i, 0)),
        pl.BlockSpec((BN, 1), lambda i: (i, 0)),
    ],
    out_shape=[
        jax.ShapeDtypeStruct((N, 1), jnp.float32),
        jax.ShapeDtypeStruct((N, 1), jnp.float32),
    ],
)


def _sage_block(h, agg_ref, invd_ref, msk_ref, ws_ref, wn_ref, wi_ref, b_ref):
    agg = agg_ref[0] + agg_ref[1]
    neigh = agg * invd_ref[...]
    acc = (
        jnp.dot(h, ws_ref[...], preferred_element_type=jnp.float32)
        + jnp.dot(neigh, wn_ref[...], preferred_element_type=jnp.float32)
        + msk_ref[...] * jnp.dot(h, wi_ref[...], preferred_element_type=jnp.float32)
        + b_ref[...]
    )
    return jnp.maximum(acc, 0.0)


def _layer_body(h_ref, agg_ref, invd_ref, msk_ref, ws_ref, wn_ref, wi_ref, b_ref, o_ref):
    o_ref[...] = _sage_block(
        h_ref[...], agg_ref, invd_ref, msk_ref, ws_ref, wn_ref, wi_ref, b_ref
    )


def _head_body(h_ref, agg_ref, invd_ref, msk_ref, ws_ref, wn_ref, wi_ref, b_ref,
               wm1_ref, bm1_ref, wm2_ref, bm2_ref, o_ref):
    h3 = _sage_block(
        h_ref[...], agg_ref, invd_ref, msk_ref, ws_ref, wn_ref, wi_ref, b_ref
    )
    t = jnp.maximum(
        jnp.dot(h3, wm1_ref[...], preferred_element_type=jnp.float32) + bm1_ref[...],
        0.0,
    )
    o_ref[...] = jnp.dot(t, wm2_ref[...], preferred_element_type=jnp.float32) + bm2_ref[...]


_node_specs = [
    pl.BlockSpec((BN, H), lambda i: (i, 0)),
    pl.BlockSpec((2, BN, H), lambda i: (0, i, 0)),
    pl.BlockSpec((BN, 1), lambda i: (i, 0)),
    pl.BlockSpec((BN, 1), lambda i: (i, 0)),
]
_w_specs = [
    pl.BlockSpec((D, H), lambda i: (0, 0)),
    pl.BlockSpec((D, H), lambda i: (0, 0)),
    pl.BlockSpec((D, H), lambda i: (0, 0)),
    pl.BlockSpec((1, H), lambda i: (0, 0)),
]

_layer_tc = pl.pallas_call(
    _layer_body,
    grid=(N // BN,),
    in_specs=_node_specs + _w_specs,
    out_specs=pl.BlockSpec((BN, H), lambda i: (i, 0)),
    out_shape=jax.ShapeDtypeStruct((N, H), jnp.float32),
)

_head_tc = pl.pallas_call(
    _head_body,
    grid=(N // BN,),
    in_specs=_node_specs + _w_specs + [
        pl.BlockSpec((H, MH), lambda i: (0, 0)),
        pl.BlockSpec((1, MH), lambda i: (0, 0)),
        pl.BlockSpec((MH, NL), lambda i: (0, 0)),
        pl.BlockSpec((1, NL), lambda i: (0, 0)),
    ],
    out_specs=pl.BlockSpec((BN, NL), lambda i: (i, 0)),
    out_shape=jax.ShapeDtypeStruct((N, NL), jnp.float32),
)


def kernel(x, unused, Ws0, Wn0, Wi0, b0, Ws1, Wn1, Wi1, b1, Ws2, Wn2, Wi2, b2,
           Wm1, bm1, Wm2, bm2, edge_index, id_index):
    src = edge_index[0].astype(jnp.int32)
    dst = edge_index[1].astype(jnp.int32)
    pad = jnp.arange(EP - E, dtype=jnp.int32)
    srcp = jnp.concatenate([src, pad % N]).reshape(NW * NBW, B)
    dstp = jnp.concatenate([dst, N + pad % (NA - N)]).reshape(NW * NBW, B)
    idp = jnp.concatenate(
        [id_index.astype(jnp.int32), jnp.full((IDP - id_index.shape[0],), N, jnp.int32)]
    )

    degw = _deg_pass(dstp).reshape(2, NA, H)[:, :N, :]
    invd, msk = _compress_tc(degw, idp.reshape(1, IDP))

    b0r = b0.reshape(1, H)
    b1r = b1.reshape(1, H)
    b2r = b2.reshape(1, H)
    bm1r = bm1.reshape(1, MH)
    bm2r = bm2.reshape(1, NL)

    h = x
    agg = _agg_pass(h, srcp, dstp).reshape(2, NA, H)[:, :N, :]
    h = _layer_tc(h, agg, invd, msk, Ws0, Wn0, Wi0, b0r)
    agg = _agg_pass(h, srcp, dstp).reshape(2, NA, H)[:, :N, :]
    h = _layer_tc(h, agg, invd, msk, Ws1, Wn1, Wi1, b1r)
    agg = _agg_pass(h, srcp, dstp).reshape(2, NA, H)[:, :N, :]
    return _head_tc(h, agg, invd, msk, Ws2, Wn2, Wi2, b2r, Wm1, bm1r, Wm2, bm2r)


# speedup vs baseline: 2.9767x; 1.0135x over previous
import functools

import jax
import jax.numpy as jnp
from jax import lax
from jax.experimental import pallas as pl
from jax.experimental.pallas import tpu as pltpu
from jax.experimental.pallas import tpu_sc as plsc

N = 10000
D = 128
H = 128
E = 320000
NL = 40
MH = 256

NC = 2
NS = 16
NW = NC * NS
B = 128
NBW = 80
CH = 16
NCH = NBW // CH
EP = NW * NBW * B
NA = 10112
RPT = NA // NS
ZCH = (128, 128, 128, 128, 120)
IDP = 1024

_mesh = plsc.VectorSubcoreMesh(
    core_axis_name="c", subcore_axis_name="s", num_cores=NC, num_subcores=NS
)


def _fill_f32(buf, rows, cols, val):
    vec = jnp.full((16,), val, jnp.float32)

    def body(i, carry):
        for k in range(cols // 16):
            buf[i, pl.ds(k * 16, 16)] = vec
        return carry

    lax.fori_loop(0, rows, body, 0)


@functools.partial(
    pl.kernel,
    out_type=jax.ShapeDtypeStruct((2 * NA, H), jnp.float32),
    mesh=_mesh,
    scratch_types=[
        pltpu.VMEM((CH, B), jnp.int32),
        pltpu.VMEM((CH, B), jnp.int32),
        pltpu.VMEM((B, H), jnp.float32),
        pltpu.VMEM((B, H), jnp.float32),
        pltpu.VMEM_SHARED((NA, H), jnp.float32),
        pltpu.SemaphoreType.DMA,
        pltpu.SemaphoreType.DMA,
    ],
)
def _agg_pass(h_hbm, srcp_hbm, dstp_hbm, out_hbm, sidx_v, didx_v, rows_a, rows_b,
              agg_sh, gsem, ssem):
    c = lax.axis_index("c")
    s = lax.axis_index("s")
    wid = c * NS + s
    _agg_phase(h_hbm, srcp_hbm, dstp_hbm, out_hbm, sidx_v, didx_v, rows_a,
               rows_b, agg_sh, gsem, ssem, c, s, wid)


def _agg_phase(h_hbm, srcp_hbm, dstp_hbm, out_hbm, sidx_v, didx_v, rows_a, rows_b,
               agg_sh, gsem, ssem, c, s, wid):
    _fill_f32(rows_a, B, H, 0.0)
    off = 0
    for sz in ZCH:
        pltpu.sync_copy(rows_a.at[pl.ds(0, sz)], agg_sh.at[pl.ds(s * RPT + off, sz)])
        off += sz
    plsc.subcore_barrier()

    def gather(j, buf):
        pltpu.async_copy(h_hbm.at[sidx_v.at[j]], buf, gsem)

    def gather_wait(buf):
        pltpu.make_async_copy(h_hbm.at[pl.ds(0, B)], buf, gsem).wait()

    def scatter(j, buf):
        pltpu.async_copy(buf, agg_sh.at[didx_v.at[j]], ssem, add=True)

    def scatter_wait(buf):
        pltpu.make_async_copy(buf, agg_sh.at[pl.ds(0, B)], ssem).wait()

    bufs = (rows_a, rows_b)

    def chunk_body(ch, carry):
        base = wid * NBW + ch * CH
        pltpu.sync_copy(srcp_hbm.at[pl.ds(base, CH)], sidx_v)
        pltpu.sync_copy(dstp_hbm.at[pl.ds(base, CH)], didx_v)
        gather(0, bufs[0])
        for jj in range(1, CH):
            cur, prv = bufs[jj % 2], bufs[(jj - 1) % 2]
            if jj >= 2:
                scatter_wait(cur)
            gather(jj, cur)
            gather_wait(prv)
            scatter(jj - 1, prv)
        last = bufs[(CH - 1) % 2]
        scatter_wait(bufs[(CH - 2) % 2])
        gather_wait(last)
        scatter(CH - 1, last)
        scatter_wait(last)
        return carry

    lax.fori_loop(0, NCH, chunk_body, 0)
    plsc.subcore_barrier()

    pltpu.sync_copy(
        agg_sh.at[pl.ds(s * RPT, RPT)], out_hbm.at[pl.ds(c * NA + s * RPT, RPT)]
    )


@functools.partial(
    pl.kernel,
    out_type=(
        jax.ShapeDtypeStruct((2 * NA, H), jnp.float32),
        jax.ShapeDtypeStruct((2 * NA, H), jnp.float32),
    ),
    mesh=_mesh,
    scratch_types=[
        pltpu.VMEM((CH, B), jnp.int32),
        pltpu.VMEM((CH, B), jnp.int32),
        pltpu.VMEM((B, H), jnp.float32),
        pltpu.VMEM((B, H), jnp.float32),
        pltpu.VMEM_SHARED((NA, H), jnp.float32),
        pltpu.SemaphoreType.DMA,
        pltpu.SemaphoreType.DMA,
    ],
)
def _agg_deg_pass(h_hbm, srcp_hbm, dstp_hbm, agg_out, deg_out, sidx_v, didx_v,
                  rows_a, rows_b, acc_sh, gsem, ssem):
    c = lax.axis_index("c")
    s = lax.axis_index("s")
    wid = c * NS + s

    _agg_phase(h_hbm, srcp_hbm, dstp_hbm, agg_out, sidx_v, didx_v, rows_a,
               rows_b, acc_sh, gsem, ssem, c, s, wid)

    _fill_f32(rows_a, B, H, 0.0)
    off = 0
    for sz in ZCH:
        pltpu.sync_copy(rows_a.at[pl.ds(0, sz)], acc_sh.at[pl.ds(s * RPT + off, sz)])
        off += sz
    _fill_f32(rows_a, B, H, 1.0)
    plsc.subcore_barrier()

    def chunk_body(ch, carry):
        pltpu.sync_copy(dstp_hbm.at[pl.ds(wid * NBW + ch * CH, CH)], didx_v)
        for jj in range(CH):
            pltpu.async_copy(rows_a, acc_sh.at[didx_v.at[jj]], ssem, add=True)
        for jj in range(CH):
            pltpu.make_async_copy(rows_a, acc_sh.at[pl.ds(0, B)], ssem).wait()
        return carry

    lax.fori_loop(0, NCH, chunk_body, 0)
    plsc.subcore_barrier()
    pltpu.sync_copy(
        acc_sh.at[pl.ds(s * RPT, RPT)], deg_out.at[pl.ds(c * NA + s * RPT, RPT)]
    )


BN = 1000


def _sage_block(h, agg_ref, invd_ref, msk_ref, ws_ref, wn_ref, wi_ref, b_ref):
    agg = agg_ref[0] + agg_ref[1]
    neigh = agg * invd_ref[...]
    acc = (
        jnp.dot(h, ws_ref[...], preferred_element_type=jnp.float32)
        + jnp.dot(neigh, wn_ref[...], preferred_element_type=jnp.float32)
        + msk_ref[...] * jnp.dot(h, wi_ref[...], preferred_element_type=jnp.float32)
        + b_ref[...]
    )
    return jnp.maximum(acc, 0.0)


def _layer_body(h_ref, agg_ref, invd_ref, msk_ref, ws_ref, wn_ref, wi_ref, b_ref, o_ref):
    o_ref[...] = _sage_block(
        h_ref[...], agg_ref, invd_ref, msk_ref, ws_ref, wn_ref, wi_ref, b_ref
    )


def _layer1_body(h_ref, agg_ref, dw_ref, ids_ref, ws_ref, wn_ref, wi_ref, b_ref,
                 o_ref, invd_ref, msk_ref):
    i = pl.program_id(0)
    d = jnp.maximum(dw_ref[0, :, 0:1] + dw_ref[1, :, 0:1], 1.0)
    invd_ref[...] = 1.0 / d
    rowid = jax.lax.broadcasted_iota(jnp.int32, (BN, IDP), 0) + i * BN
    hit = rowid == ids_ref[...]
    msk_ref[...] = jnp.any(hit, axis=1, keepdims=True).astype(jnp.float32)
    o_ref[...] = _sage_block(
        h_ref[...], agg_ref, invd_ref, msk_ref, ws_ref, wn_ref, wi_ref, b_ref
    )


def _head_body(h_ref, agg_ref, invd_ref, msk_ref, ws_ref, wn_ref, wi_ref, b_ref,
               wm1_ref, bm1_ref, wm2_ref, bm2_ref, o_ref):
    h3 = _sage_block(
        h_ref[...], agg_ref, invd_ref, msk_ref, ws_ref, wn_ref, wi_ref, b_ref
    )
    t = jnp.maximum(
        jnp.dot(h3, wm1_ref[...], preferred_element_type=jnp.float32) + bm1_ref[...],
        0.0,
    )
    o_ref[...] = jnp.dot(t, wm2_ref[...], preferred_element_type=jnp.float32) + bm2_ref[...]


_node_specs = [
    pl.BlockSpec((BN, H), lambda i: (i, 0)),
    pl.BlockSpec((2, BN, H), lambda i: (0, i, 0)),
    pl.BlockSpec((BN, 1), lambda i: (i, 0)),
    pl.BlockSpec((BN, 1), lambda i: (i, 0)),
]
_w_specs = [
    pl.BlockSpec((D, H), lambda i: (0, 0)),
    pl.BlockSpec((D, H), lambda i: (0, 0)),
    pl.BlockSpec((D, H), lambda i: (0, 0)),
    pl.BlockSpec((1, H), lambda i: (0, 0)),
]

_layer_tc = pl.pallas_call(
    _layer_body,
    grid=(N // BN,),
    in_specs=_node_specs + _w_specs,
    out_specs=pl.BlockSpec((BN, H), lambda i: (i, 0)),
    out_shape=jax.ShapeDtypeStruct((N, H), jnp.float32),
)

_layer1_tc = pl.pallas_call(
    _layer1_body,
    grid=(N // BN,),
    in_specs=[
        pl.BlockSpec((BN, H), lambda i: (i, 0)),
        pl.BlockSpec((2, BN, H), lambda i: (0, i, 0)),
        pl.BlockSpec((2, BN, H), lambda i: (0, i, 0)),
        pl.BlockSpec((1, IDP), lambda i: (0, 0)),
    ] + _w_specs,
    out_specs=[
        pl.BlockSpec((BN, H), lambda i: (i, 0)),
        pl.BlockSpec((BN, 1), lambda i: (i, 0)),
        pl.BlockSpec((BN, 1), lambda i: (i, 0)),
    ],
    out_shape=[
        jax.ShapeDtypeStruct((N, H), jnp.float32),
        jax.ShapeDtypeStruct((N, 1), jnp.float32),
        jax.ShapeDtypeStruct((N, 1), jnp.float32),
    ],
)

_head_tc = pl.pallas_call(
    _head_body,
    grid=(N // BN,),
    in_specs=_node_specs + _w_specs + [
        pl.BlockSpec((H, MH), lambda i: (0, 0)),
        pl.BlockSpec((1, MH), lambda i: (0, 0)),
        pl.BlockSpec((MH, NL), lambda i: (0, 0)),
        pl.BlockSpec((1, NL), lambda i: (0, 0)),
    ],
    out_specs=pl.BlockSpec((BN, NL), lambda i: (i, 0)),
    out_shape=jax.ShapeDtypeStruct((N, NL), jnp.float32),
)


def kernel(x, unused, Ws0, Wn0, Wi0, b0, Ws1, Wn1, Wi1, b1, Ws2, Wn2, Wi2, b2,
           Wm1, bm1, Wm2, bm2, edge_index, id_index):
    src = edge_index[0].astype(jnp.int32)
    dst = edge_index[1].astype(jnp.int32)
    pad = jnp.arange(EP - E, dtype=jnp.int32)
    srcp = jnp.concatenate([src, pad % N]).reshape(NW * NBW, B)
    dstp = jnp.concatenate([dst, N + pad % (NA - N)]).reshape(NW * NBW, B)
    idp = jnp.concatenate(
        [id_index.astype(jnp.int32), jnp.full((IDP - id_index.shape[0],), N, jnp.int32)]
    )

    b0r = b0.reshape(1, H)
    b1r = b1.reshape(1, H)
    b2r = b2.reshape(1, H)
    bm1r = bm1.reshape(1, MH)
    bm2r = bm2.reshape(1, NL)

    aggf, degf = _agg_deg_pass(x, srcp, dstp)
    agg = aggf.reshape(2, NA, H)[:, :N, :]
    degw = degf.reshape(2, NA, H)[:, :N, :]
    h, invd, msk = _layer1_tc(x, agg, degw, idp.reshape(1, IDP), Ws0, Wn0, Wi0, b0r)
    agg = _agg_pass(h, srcp, dstp).reshape(2, NA, H)[:, :N, :]
    h = _layer_tc(h, agg, invd, msk, Ws1, Wn1, Wi1, b1r)
    agg = _agg_pass(h, srcp, dstp).reshape(2, NA, H)[:, :N, :]
    return _head_tc(h, agg, invd, msk, Ws2, Wn2, Wi2, b2r, Wm1, bm1r, Wm2, bm2r)


# speedup vs baseline: 3.1505x; 1.0584x over previous
import functools

import jax
import jax.numpy as jnp
from jax import lax
from jax.experimental import pallas as pl
from jax.experimental.pallas import tpu as pltpu
from jax.experimental.pallas import tpu_sc as plsc

N = 10000
D = 128
H = 128
E = 320000
NL = 40
MH = 256

NC = 2
NS = 16
NW = NC * NS
B = 128
NBW = 80
CH = 8
NCH = NBW // CH
EP = NW * NBW * B
NA = 10112
RPT = NA // NS
ZCH = (128, 128, 128, 128, 120)
IDP = 1024

_mesh = plsc.VectorSubcoreMesh(
    core_axis_name="c", subcore_axis_name="s", num_cores=NC, num_subcores=NS
)


def _fill_f32(buf, rows, cols, val):
    vec = jnp.full((16,), val, jnp.float32)

    def body(i, carry):
        for k in range(cols // 16):
            buf[i, pl.ds(k * 16, 16)] = vec
        return carry

    lax.fori_loop(0, rows, body, 0)


@functools.partial(
    pl.kernel,
    out_type=jax.ShapeDtypeStruct((2 * NA, H), jnp.float32),
    mesh=_mesh,
    scratch_types=[
        pltpu.VMEM((CH, B), jnp.int32),
        pltpu.VMEM((CH, B), jnp.int32),
        pltpu.VMEM((CH, B), jnp.int32),
        pltpu.VMEM((CH, B), jnp.int32),
        pltpu.VMEM((B, H), jnp.float32),
        pltpu.VMEM((B, H), jnp.float32),
        pltpu.VMEM_SHARED((NA, H), jnp.float32),
        pltpu.SemaphoreType.DMA,
        pltpu.SemaphoreType.DMA,
        pltpu.SemaphoreType.DMA,
    ],
)
def _agg_pass(h_hbm, srcp_hbm, dstp_hbm, out_hbm, sidx0, sidx1, didx0, didx1,
              rows_a, rows_b, agg_sh, gsem, ssem, isem):
    c = lax.axis_index("c")
    s = lax.axis_index("s")
    wid = c * NS + s
    _agg_phase(h_hbm, srcp_hbm, dstp_hbm, out_hbm, (sidx0, sidx1),
               (didx0, didx1), (rows_a, rows_b), agg_sh, gsem, ssem, isem,
               c, s, wid)


def _agg_phase(h_hbm, srcp_hbm, dstp_hbm, out_hbm, sidx, didx, bufs, agg_sh,
               gsem, ssem, isem, c, s, wid):
    base = wid * NBW

    def pf_idx(ch):
        p = ch % 2
        pltpu.async_copy(srcp_hbm.at[pl.ds(base + ch * CH, CH)], sidx[p], isem)
        pltpu.async_copy(dstp_hbm.at[pl.ds(base + ch * CH, CH)], didx[p], isem)

    def pf_wait(ch):
        p = ch % 2
        pltpu.make_async_copy(srcp_hbm.at[pl.ds(0, CH)], sidx[p], isem).wait()
        pltpu.make_async_copy(dstp_hbm.at[pl.ds(0, CH)], didx[p], isem).wait()

    pf_idx(0)

    _fill_f32(bufs[0], B, H, 0.0)
    off = 0
    for sz in ZCH:
        pltpu.sync_copy(bufs[0].at[pl.ds(0, sz)], agg_sh.at[pl.ds(s * RPT + off, sz)])
        off += sz
    plsc.subcore_barrier()

    def gather(g, buf):
        pltpu.async_copy(h_hbm.at[sidx[(g // CH) % 2].at[g % CH]], buf, gsem)

    def gather_wait(buf):
        pltpu.make_async_copy(h_hbm.at[pl.ds(0, B)], buf, gsem).wait()

    def scatter(g, buf):
        pltpu.async_copy(buf, agg_sh.at[didx[(g // CH) % 2].at[g % CH]], ssem,
                         add=True)

    def scatter_wait(buf):
        pltpu.make_async_copy(buf, agg_sh.at[pl.ds(0, B)], ssem).wait()

    pf_wait(0)
    pf_idx(1)
    gather(0, bufs[0])
    for g in range(1, NBW):
        cur, prv = bufs[g % 2], bufs[(g - 1) % 2]
        if g % CH == 0:
            pf_wait(g // CH)
        if g >= 2:
            scatter_wait(cur)
        gather(g, cur)
        gather_wait(prv)
        scatter(g - 1, prv)
        if g % CH == 2 and g < NBW - CH:
            pf_idx(g // CH + 1)
    last = bufs[(NBW - 1) % 2]
    scatter_wait(bufs[(NBW - 2) % 2])
    gather_wait(last)
    scatter(NBW - 1, last)
    scatter_wait(last)
    plsc.subcore_barrier()

    pltpu.sync_copy(
        agg_sh.at[pl.ds(s * RPT, RPT)], out_hbm.at[pl.ds(c * NA + s * RPT, RPT)]
    )


@functools.partial(
    pl.kernel,
    out_type=(
        jax.ShapeDtypeStruct((2 * NA, H), jnp.float32),
        jax.ShapeDtypeStruct((2 * NA, H), jnp.float32),
    ),
    mesh=_mesh,
    scratch_types=[
        pltpu.VMEM((CH, B), jnp.int32),
        pltpu.VMEM((CH, B), jnp.int32),
        pltpu.VMEM((CH, B), jnp.int32),
        pltpu.VMEM((CH, B), jnp.int32),
        pltpu.VMEM((B, H), jnp.float32),
        pltpu.VMEM((B, H), jnp.float32),
        pltpu.VMEM_SHARED((NA, H), jnp.float32),
        pltpu.SemaphoreType.DMA,
        pltpu.SemaphoreType.DMA,
        pltpu.SemaphoreType.DMA,
    ],
)
def _agg_deg_pass(h_hbm, srcp_hbm, dstp_hbm, agg_out, deg_out, sidx0, sidx1,
                  didx0, didx1, rows_a, rows_b, acc_sh, gsem, ssem, isem):
    c = lax.axis_index("c")
    s = lax.axis_index("s")
    wid = c * NS + s
    didx = (didx0, didx1)
    base = wid * NBW

    _agg_phase(h_hbm, srcp_hbm, dstp_hbm, agg_out, (sidx0, sidx1), didx,
               (rows_a, rows_b), acc_sh, gsem, ssem, isem, c, s, wid)

    def pf_idx(ch):
        pltpu.async_copy(dstp_hbm.at[pl.ds(base + ch * CH, CH)], didx[ch % 2], isem)

    def pf_wait(ch):
        pltpu.make_async_copy(dstp_hbm.at[pl.ds(0, CH)], didx[ch % 2], isem).wait()

    pf_idx(0)
    _fill_f32(rows_a, B, H, 0.0)
    off = 0
    for sz in ZCH:
        pltpu.sync_copy(rows_a.at[pl.ds(0, sz)], acc_sh.at[pl.ds(s * RPT + off, sz)])
        off += sz
    _fill_f32(rows_a, B, H, 1.0)
    plsc.subcore_barrier()

    for ch in range(NCH):
        pf_wait(ch)
        if ch + 1 < NCH:
            pf_idx(ch + 1)
        for jj in range(CH):
            pltpu.async_copy(rows_a, acc_sh.at[didx[ch % 2].at[jj]], ssem, add=True)
        for jj in range(CH):
            pltpu.make_async_copy(rows_a, acc_sh.at[pl.ds(0, B)], ssem).wait()
    plsc.subcore_barrier()
    pltpu.sync_copy(
        acc_sh.at[pl.ds(s * RPT, RPT)], deg_out.at[pl.ds(c * NA + s * RPT, RPT)]
    )


BN = 1000


def _sage_block(h, agg_ref, invd_ref, msk_ref, ws_ref, wn_ref, wi_ref, b_ref):
    agg = agg_ref[0] + agg_ref[1]
    neigh = agg * invd_ref[...]
    acc = (
        jnp.dot(h, ws_ref[...], preferred_element_type=jnp.float32)
        + jnp.dot(neigh, wn_ref[...], preferred_element_type=jnp.float32)
        + msk_ref[...] * jnp.dot(h, wi_ref[...], preferred_element_type=jnp.float32)
        + b_ref[...]
    )
    return jnp.maximum(acc, 0.0)


def _layer_body(h_ref, agg_ref, invd_ref, msk_ref, ws_ref, wn_ref, wi_ref, b_ref, o_ref):
    o_ref[...] = _sage_block(
        h_ref[...], agg_ref, invd_ref, msk_ref, ws_ref, wn_ref, wi_ref, b_ref
    )


def _layer1_body(h_ref, agg_ref, dw_ref, ids_ref, ws_ref, wn_ref, wi_ref, b_ref,
                 o_ref, invd_ref, msk_ref):
    i = pl.program_id(0)
    d = jnp.maximum(dw_ref[0, :, 0:1] + dw_ref[1, :, 0:1], 1.0)
    invd_ref[...] = 1.0 / d
    rowid = jax.lax.broadcasted_iota(jnp.int32, (BN, IDP), 0) + i * BN
    hit = rowid == ids_ref[...]
    msk_ref[...] = jnp.any(hit, axis=1, keepdims=True).astype(jnp.float32)
    o_ref[...] = _sage_block(
        h_ref[...], agg_ref, invd_ref, msk_ref, ws_ref, wn_ref, wi_ref, b_ref
    )


def _head_body(h_ref, agg_ref, invd_ref, msk_ref, ws_ref, wn_ref, wi_ref, b_ref,
               wm1_ref, bm1_ref, wm2_ref, bm2_ref, o_ref):
    h3 = _sage_block(
        h_ref[...], agg_ref, invd_ref, msk_ref, ws_ref, wn_ref, wi_ref, b_ref
    )
    t = jnp.maximum(
        jnp.dot(h3, wm1_ref[...], preferred_element_type=jnp.float32) + bm1_ref[...],
        0.0,
    )
    o_ref[...] = jnp.dot(t, wm2_ref[...], preferred_element_type=jnp.float32) + bm2_ref[...]


_node_specs = [
    pl.BlockSpec((BN, H), lambda i: (i, 0)),
    pl.BlockSpec((2, BN, H), lambda i: (0, i, 0)),
    pl.BlockSpec((BN, 1), lambda i: (i, 0)),
    pl.BlockSpec((BN, 1), lambda i: (i, 0)),
]
_w_specs = [
    pl.BlockSpec((D, H), lambda i: (0, 0)),
    pl.BlockSpec((D, H), lambda i: (0, 0)),
    pl.BlockSpec((D, H), lambda i: (0, 0)),
    pl.BlockSpec((1, H), lambda i: (0, 0)),
]

_layer_tc = pl.pallas_call(
    _layer_body,
    grid=(N // BN,),
    in_specs=_node_specs + _w_specs,
    out_specs=pl.BlockSpec((BN, H), lambda i: (i, 0)),
    out_shape=jax.ShapeDtypeStruct((N, H), jnp.float32),
)

_layer1_tc = pl.pallas_call(
    _layer1_body,
    grid=(N // BN,),
    in_specs=[
        pl.BlockSpec((BN, H), lambda i: (i, 0)),
        pl.BlockSpec((2, BN, H), lambda i: (0, i, 0)),
        pl.BlockSpec((2, BN, H), lambda i: (0, i, 0)),
        pl.BlockSpec((1, IDP), lambda i: (0, 0)),
    ] + _w_specs,
    out_specs=[
        pl.BlockSpec((BN, H), lambda i: (i, 0)),
        pl.BlockSpec((BN, 1), lambda i: (i, 0)),
        pl.BlockSpec((BN, 1), lambda i: (i, 0)),
    ],
    out_shape=[
        jax.ShapeDtypeStruct((N, H), jnp.float32),
        jax.ShapeDtypeStruct((N, 1), jnp.float32),
        jax.ShapeDtypeStruct((N, 1), jnp.float32),
    ],
)

_head_tc = pl.pallas_call(
    _head_body,
    grid=(N // BN,),
    in_specs=_node_specs + _w_specs + [
        pl.BlockSpec((H, MH), lambda i: (0, 0)),
        pl.BlockSpec((1, MH), lambda i: (0, 0)),
        pl.BlockSpec((MH, NL), lambda i: (0, 0)),
        pl.BlockSpec((1, NL), lambda i: (0, 0)),
    ],
    out_specs=pl.BlockSpec((BN, NL), lambda i: (i, 0)),
    out_shape=jax.ShapeDtypeStruct((N, NL), jnp.float32),
)


def kernel(x, unused, Ws0, Wn0, Wi0, b0, Ws1, Wn1, Wi1, b1, Ws2, Wn2, Wi2, b2,
           Wm1, bm1, Wm2, bm2, edge_index, id_index):
    src = edge_index[0].astype(jnp.int32)
    dst = edge_index[1].astype(jnp.int32)
    pad = jnp.arange(EP - E, dtype=jnp.int32)
    srcp = jnp.concatenate([src, pad % N]).reshape(NW * NBW, B)
    dstp = jnp.concatenate([dst, N + pad % (NA - N)]).reshape(NW * NBW, B)
    idp = jnp.concatenate(
        [id_index.astype(jnp.int32), jnp.full((IDP - id_index.shape[0],), N, jnp.int32)]
    )

    b0r = b0.reshape(1, H)
    b1r = b1.reshape(1, H)
    b2r = b2.reshape(1, H)
    bm1r = bm1.reshape(1, MH)
    bm2r = bm2.reshape(1, NL)

    aggf, degf = _agg_deg_pass(x, srcp, dstp)
    agg = aggf.reshape(2, NA, H)[:, :N, :]
    degw = degf.reshape(2, NA, H)[:, :N, :]
    h, invd, msk = _layer1_tc(x, agg, degw, idp.reshape(1, IDP), Ws0, Wn0, Wi0, b0r)
    agg = _agg_pass(h, srcp, dstp).reshape(2, NA, H)[:, :N, :]
    h = _layer_tc(h, agg, invd, msk, Ws1, Wn1, Wi1, b1r)
    agg = _agg_pass(h, srcp, dstp).reshape(2, NA, H)[:, :N, :]
    return _head_tc(h, agg, invd, msk, Ws2, Wn2, Wi2, b2r, Wm1, bm1r, Wm2, bm2r)
